# Initial kernel scaffold; baseline (speedup 1.0000x reference)
import jax
import jax.numpy as jnp
from jax.experimental import pallas as pl


def kernel(indices, table):
    raise NotImplementedError("write your pallas kernel here")


# speedup vs baseline: 23.1511x; 23.1511x over previous
import functools

import jax
import jax.numpy as jnp
from jax import lax
from jax.experimental import pallas as pl
from jax.experimental.pallas import tpu as pltpu
from jax.experimental.pallas import tpu_sc as plsc


def _make_gather(V, D, B, nc, ns, chunk):
    nw = nc * ns
    per_w = B // nw
    steps = per_w // chunk
    mesh = plsc.VectorSubcoreMesh(core_axis_name="c", subcore_axis_name="s")

    @functools.partial(
        pl.kernel,
        out_type=jax.ShapeDtypeStruct((B, D), jnp.float32),
        mesh=mesh,
        scratch_types=[
            pltpu.VMEM((chunk,), jnp.int32),
            pltpu.VMEM((chunk, D), jnp.float32),
            pltpu.SemaphoreType.DMA,
        ],
        compiler_params=pltpu.CompilerParams(use_tc_tiling_on_sc=False),
    )
    def gather_kernel(table_hbm, idx_hbm, out_hbm, idx_v, rows_v, sem):
        wid = lax.axis_index("s") * nc + lax.axis_index("c")
        base = wid * per_w

        def body(i, _):
            off = base + i * chunk
            pltpu.sync_copy(idx_hbm.at[pl.ds(off, chunk)], idx_v)
            pltpu.async_copy(table_hbm.at[idx_v], rows_v, sem).wait()
            pltpu.sync_copy(rows_v, out_hbm.at[pl.ds(off, chunk)])
            return 0

        lax.fori_loop(0, steps, body, 0, unroll=False)

    return gather_kernel


def kernel(indices, table):
    B0, S = indices.shape
    V, D = table.shape
    B = B0 * S
    info = plsc.get_sparse_core_info()
    nc, ns = info.num_cores, info.num_subcores
    flat_idx = indices.reshape(B).astype(jnp.int32)
    gather = _make_gather(V, D, B, nc, ns, chunk=4096)
    out = gather(table, flat_idx)
    return out.reshape(B0, S, D)


# speedup vs baseline: 23.3759x; 1.0097x over previous
import functools

import jax
import jax.numpy as jnp
from jax import lax
from jax.experimental import pallas as pl
from jax.experimental.pallas import tpu as pltpu
from jax.experimental.pallas import tpu_sc as plsc


def _make_gather(V, D, B, nc, ns, chunk):
    nw = nc * ns
    per_w = B // nw
    steps = per_w // chunk
    mesh = plsc.VectorSubcoreMesh(core_axis_name="c", subcore_axis_name="s")

    @functools.partial(
        pl.kernel,
        out_type=jax.ShapeDtypeStruct((B, D), jnp.float32),
        mesh=mesh,
        scratch_types=[
            pltpu.VMEM((chunk,), jnp.int32),
            pltpu.VMEM((chunk, D), jnp.float32),
            pltpu.SemaphoreType.DMA,
        ],
        compiler_params=pltpu.CompilerParams(use_tc_tiling_on_sc=False),
    )
    def gather_kernel(table_hbm, idx_hbm, out_hbm, idx_v, rows_v, sem):
        wid = lax.axis_index("s") * nc + lax.axis_index("c")
        base = wid * per_w

        def body(i, _):
            off = base + i * chunk
            pltpu.sync_copy(idx_hbm.at[pl.ds(off, chunk)], idx_v)
            pltpu.async_copy(table_hbm.at[idx_v], rows_v, sem).wait()
            pltpu.sync_copy(rows_v, out_hbm.at[pl.ds(off, chunk)])
            return 0

        lax.fori_loop(0, steps, body, 0, unroll=False)

    return gather_kernel


def kernel(indices, table):
    B0, S = indices.shape
    V, D = table.shape
    B = B0 * S
    info = plsc.get_sparse_core_info()
    nc, ns = info.num_cores, info.num_subcores
    flat_idx = indices.reshape(B).astype(jnp.int32)
    gather = _make_gather(V, D, B, nc, ns, chunk=10240)
    out = gather(table, flat_idx)
    return out.reshape(B0, S, D)


# speedup vs baseline: 23.5583x; 1.0078x over previous
import functools

import jax
import jax.numpy as jnp
from jax import lax
from jax.experimental import pallas as pl
from jax.experimental.pallas import tpu as pltpu
from jax.experimental.pallas import tpu_sc as plsc

_NBUF = 4


def _make_gather(V, D, B, nc, ns, chunk):
    nw = nc * ns
    per_w = B // nw
    steps = per_w // chunk
    groups = steps // _NBUF
    mesh = plsc.VectorSubcoreMesh(core_axis_name="c", subcore_axis_name="s")

    @functools.partial(
        pl.kernel,
        out_type=jax.ShapeDtypeStruct((B, D), jnp.float32),
        mesh=mesh,
        scratch_types=(
            [pltpu.VMEM((chunk,), jnp.int32) for _ in range(_NBUF)]
            + [pltpu.VMEM((chunk, D), jnp.float32) for _ in range(_NBUF)]
            + [pltpu.SemaphoreType.DMA for _ in range(2 * _NBUF)]
        ),
        compiler_params=pltpu.CompilerParams(use_tc_tiling_on_sc=False),
    )
    def gather_kernel(table_hbm, idx_hbm, out_hbm, *scratch):
        idx_v = scratch[:_NBUF]
        rows_v = scratch[_NBUF:2 * _NBUF]
        sem_g = scratch[2 * _NBUF:3 * _NBUF]
        sem_o = scratch[3 * _NBUF:]
        wid = lax.axis_index("s") * nc + lax.axis_index("c")
        base = wid * per_w

        def body(g, _):
            offs = [base + (g * _NBUF + b) * chunk for b in range(_NBUF)]
            for b in range(_NBUF):
                @pl.when(g > 0)
                def _drain(b=b):
                    pltpu.make_async_copy(
                        rows_v[b], out_hbm.at[pl.ds(base, chunk)], sem_o[b]
                    ).wait()
                pltpu.sync_copy(idx_hbm.at[pl.ds(offs[b], chunk)], idx_v[b])
                pltpu.async_copy(table_hbm.at[idx_v[b]], rows_v[b], sem_g[b])
            for b in range(_NBUF):
                pltpu.make_async_copy(
                    table_hbm.at[idx_v[b]], rows_v[b], sem_g[b]
                ).wait()
                pltpu.async_copy(
                    rows_v[b], out_hbm.at[pl.ds(offs[b], chunk)], sem_o[b]
                )
            return 0

        lax.fori_loop(0, groups, body, 0, unroll=False)
        for b in range(_NBUF):
            pltpu.make_async_copy(
                rows_v[b], out_hbm.at[pl.ds(base, chunk)], sem_o[b]
            ).wait()

    return gather_kernel


def kernel(indices, table):
    B0, S = indices.shape
    V, D = table.shape
    B = B0 * S
    info = plsc.get_sparse_core_info()
    nc, ns = info.num_cores, info.num_subcores
    flat_idx = indices.reshape(B).astype(jnp.int32)
    gather = _make_gather(V, D, B, nc, ns, chunk=2560)
    out = gather(table, flat_idx)
    return out.reshape(B0, S, D)


# speedup vs baseline: 71.2073x; 3.0226x over previous
import functools

import jax
import jax.numpy as jnp
from jax import lax
from jax.experimental import pallas as pl
from jax.experimental.pallas import tpu as pltpu
from jax.experimental.pallas import tpu_sc as plsc

_NBUF = 4
_LB = 128


def _make_gather(V, D, B, nc, ns, ck):
    nw = nc * ns
    nblocks = B // _LB
    per_w = nblocks // nw
    steps = per_w // ck
    groups = steps // _NBUF
    ch = ck * _LB
    ob = ck * _LB * D
    mesh = plsc.VectorSubcoreMesh(core_axis_name="c", subcore_axis_name="s")

    @functools.partial(
        pl.kernel,
        out_type=jax.ShapeDtypeStruct((B * D,), jnp.float32),
        mesh=mesh,
        scratch_types=(
            [pltpu.VMEM((ch,), jnp.int32) for _ in range(_NBUF)]
            + [pltpu.VMEM((ch, D), jnp.float32) for _ in range(_NBUF)]
            + [pltpu.VMEM((ob,), jnp.float32) for _ in range(_NBUF)]
            + [pltpu.SemaphoreType.DMA for _ in range(2 * _NBUF)]
        ),
        compiler_params=pltpu.CompilerParams(
            use_tc_tiling_on_sc=False, needs_layout_passes=False
        ),
    )
    def gather_kernel(table_hbm, idx_hbm, out_hbm, *scratch):
        idx_v = scratch[:_NBUF]
        rows_v = scratch[_NBUF:2 * _NBUF]
        out_v = scratch[2 * _NBUF:3 * _NBUF]
        sem_g = scratch[3 * _NBUF:4 * _NBUF]
        sem_o = scratch[4 * _NBUF:]
        wid = lax.axis_index("s") * nc + lax.axis_index("c")
        wbase = wid * per_w
        lane = lax.iota(jnp.int32, 16)

        def body(g, _):
            k0s = [wbase + (g * _NBUF + b) * ck for b in range(_NBUF)]
            for b in range(_NBUF):
                @pl.when(g > 0)
                def _drain(b=b):
                    pltpu.make_async_copy(
                        out_v[b], out_hbm.at[pl.ds(0, ob)], sem_o[b]
                    ).wait()
                pltpu.sync_copy(idx_hbm.at[pl.ds(k0s[b] * _LB, ch)], idx_v[b])
                pltpu.async_copy(table_hbm.at[idx_v[b]], rows_v[b], sem_g[b])
            for b in range(_NBUF):
                pltpu.make_async_copy(
                    table_hbm.at[idx_v[b]], rows_v[b], sem_g[b]
                ).wait()

                def tbody(blk, _, b=b):
                    for d in range(D):
                        col = jnp.full((16,), d, jnp.int32)
                        for g2 in range(_LB // 16):
                            row = lane + (blk * _LB + g2 * 16)
                            v = plsc.load_gather(rows_v[b], [row, col])
                            out_v[b][
                                pl.ds(blk * _LB * D + d * _LB + g2 * 16, 16)
                            ] = v
                    return 0

                lax.fori_loop(0, ck, tbody, 0, unroll=False)
                pltpu.async_copy(
                    out_v[b], out_hbm.at[pl.ds(k0s[b] * _LB * D, ob)], sem_o[b]
                )
            return 0

        lax.fori_loop(0, groups, body, 0, unroll=False)
        for b in range(_NBUF):
            pltpu.make_async_copy(
                out_v[b], out_hbm.at[pl.ds(0, ob)], sem_o[b]
            ).wait()

    return gather_kernel


def kernel(indices, table):
    B0, S = indices.shape
    V, D = table.shape
    B = B0 * S
    nb = B0 // _LB
    info = plsc.get_sparse_core_info()
    nc, ns = info.num_cores, info.num_subcores
    tidx = indices.T.reshape(B).astype(jnp.int32)
    gather = _make_gather(V, D, B, nc, ns, ck=10)
    o = gather(table, tidx)
    return (
        o.reshape(S, nb, D, _LB)
        .transpose(1, 3, 0, 2)
        .reshape(B0, S, D)
    )


# speedup vs baseline: 106.7352x; 1.4989x over previous
import functools

import jax
import jax.numpy as jnp
from jax import lax
from jax.experimental import pallas as pl
from jax.experimental.pallas import tpu as pltpu
from jax.experimental.pallas import tpu_sc as plsc

_NBUF = 4
_LB = 128


def _make_format(D, VP, nc, ns, kk):
    nw = nc * ns
    nblocks = VP // _LB
    per_w = nblocks // nw
    steps = per_w // kk
    groups = steps // 2
    bw = _LB * D
    ch = kk * bw
    mesh = plsc.VectorSubcoreMesh(core_axis_name="c", subcore_axis_name="s")

    @functools.partial(
        pl.kernel,
        out_type=jax.ShapeDtypeStruct((VP * D,), jnp.float32),
        mesh=mesh,
        scratch_types=(
            [pltpu.VMEM((kk, _LB * D), jnp.float32) for _ in range(2)]
            + [pltpu.VMEM((ch,), jnp.float32) for _ in range(2)]
            + [pltpu.SemaphoreType.DMA for _ in range(4)]
        ),
        compiler_params=pltpu.CompilerParams(
            use_tc_tiling_on_sc=False, needs_layout_passes=False
        ),
    )
    def format_kernel(t3_hbm, out_hbm, *scratch):
        in_v = scratch[:2]
        out_v = scratch[2:4]
        sem_i = scratch[4:6]
        sem_o = scratch[6:]
        wid = lax.axis_index("s") * nc + lax.axis_index("c")
        wbase = wid * per_w
        lane = lax.iota(jnp.int32, 16)
        pat = (lane % D) * _LB + lane // D

        def body(g, _):
            k0s = [wbase + (g * 2 + b) * kk for b in range(2)]
            for b in range(2):
                @pl.when(g > 0)
                def _drain(b=b):
                    pltpu.make_async_copy(
                        out_v[b], out_hbm.at[pl.ds(0, ch)], sem_o[b]
                    ).wait()
                pltpu.async_copy(
                    t3_hbm.at[pl.ds(k0s[b], kk), :], in_v[b], sem_i[b]
                )
            for b in range(2):
                pltpu.make_async_copy(
                    t3_hbm.at[pl.ds(k0s[b], kk), :], in_v[b], sem_i[b]
                ).wait()

                def tbody(blk, _, b=b):
                    row = lane * 0 + blk
                    for q in range(bw // 16):
                        col = pat + q * 2
                        v = plsc.load_gather(in_v[b], [row, col])
                        out_v[b][pl.ds(blk * bw + q * 16, 16)] = v
                    return 0

                lax.fori_loop(0, kk, tbody, 0, unroll=False)
                pltpu.async_copy(
                    out_v[b], out_hbm.at[pl.ds(k0s[b] * bw, ch)], sem_o[b]
                )
            return 0

        lax.fori_loop(0, groups, body, 0, unroll=False)
        for b in range(2):
            pltpu.make_async_copy(
                out_v[b], out_hbm.at[pl.ds(0, ch)], sem_o[b]
            ).wait()

    return format_kernel


def _make_gather(VP, D, B, nc, ns, ck):
    nw = nc * ns
    nblocks = B // _LB
    per_w = nblocks // nw
    steps = per_w // ck
    groups = steps // _NBUF
    ch = ck * _LB
    ob = ck * _LB * D
    mesh = plsc.VectorSubcoreMesh(core_axis_name="c", subcore_axis_name="s")

    @functools.partial(
        pl.kernel,
        out_type=jax.ShapeDtypeStruct((B * D,), jnp.float32),
        mesh=mesh,
        scratch_types=(
            [pltpu.VMEM((ch,), jnp.int32) for _ in range(_NBUF)]
            + [pltpu.VMEM((ch, D), jnp.float32) for _ in range(_NBUF)]
            + [pltpu.VMEM((ob,), jnp.float32) for _ in range(_NBUF)]
            + [pltpu.SemaphoreType.DMA for _ in range(2 * _NBUF)]
        ),
        compiler_params=pltpu.CompilerParams(
            use_tc_tiling_on_sc=False, needs_layout_passes=False
        ),
    )
    def gather_kernel(table_hbm, idx_hbm, out_hbm, *scratch):
        idx_v = scratch[:_NBUF]
        rows_v = scratch[_NBUF:2 * _NBUF]
        out_v = scratch[2 * _NBUF:3 * _NBUF]
        sem_g = scratch[3 * _NBUF:4 * _NBUF]
        sem_o = scratch[4 * _NBUF:]
        wid = lax.axis_index("s") * nc + lax.axis_index("c")
        wbase = wid * per_w
        lane = lax.iota(jnp.int32, 16)

        def body(g, _):
            k0s = [wbase + (g * _NBUF + b) * ck for b in range(_NBUF)]
            for b in range(_NBUF):
                @pl.when(g > 0)
                def _drain(b=b):
                    pltpu.make_async_copy(
                        out_v[b], out_hbm.at[pl.ds(0, ob)], sem_o[b]
                    ).wait()
                pltpu.sync_copy(idx_hbm.at[pl.ds(k0s[b] * _LB, ch)], idx_v[b])
                pltpu.async_copy(table_hbm.at[idx_v[b]], rows_v[b], sem_g[b])
            for b in range(_NBUF):
                pltpu.make_async_copy(
                    table_hbm.at[idx_v[b]], rows_v[b], sem_g[b]
                ).wait()

                def tbody(blk, _, b=b):
                    for d in range(D):
                        col = jnp.full((16,), d, jnp.int32)
                        for g2 in range(_LB // 16):
                            row = lane + (blk * _LB + g2 * 16)
                            v = plsc.load_gather(rows_v[b], [row, col])
                            out_v[b][
                                pl.ds(blk * _LB * D + d * _LB + g2 * 16, 16)
                            ] = v
                    return 0

                lax.fori_loop(0, ck, tbody, 0, unroll=False)
                pltpu.async_copy(
                    out_v[b], out_hbm.at[pl.ds(k0s[b] * _LB * D, ob)], sem_o[b]
                )
            return 0

        lax.fori_loop(0, groups, body, 0, unroll=False)
        for b in range(_NBUF):
            pltpu.make_async_copy(
                out_v[b], out_hbm.at[pl.ds(0, ob)], sem_o[b]
            ).wait()

    return gather_kernel


def kernel(indices, table):
    B0, S = indices.shape
    V, D = table.shape
    B = B0 * S
    nb = B0 // _LB
    VP = 1 << 20
    info = plsc.get_sparse_core_info()
    nc, ns = info.num_cores, info.num_subcores

    tablep = jnp.pad(table, ((0, VP - V), (0, 0)))
    t3 = (
        tablep.T.reshape(D, VP // _LB, _LB)
        .transpose(1, 0, 2)
        .reshape(VP // _LB, _LB * D)
    )
    fmt = _make_format(D, VP, nc, ns, kk=16)
    t_rm = fmt(t3).reshape(VP, D)

    tidx = indices.T.reshape(B).astype(jnp.int32)
    gather = _make_gather(VP, D, B, nc, ns, ck=10)
    o = gather(t_rm, tidx)
    return (
        o.reshape(S, nb, D, _LB)
        .transpose(1, 3, 0, 2)
        .reshape(B0, S, D)
    )


# speedup vs baseline: 122.2583x; 1.1454x over previous
import functools

import jax
import jax.numpy as jnp
from jax import lax
from jax.experimental import pallas as pl
from jax.experimental.pallas import tpu as pltpu
from jax.experimental.pallas import tpu_sc as plsc

_NBUF = 4
_LB = 128


def _make_format(D, VP, nc, ns, kk):
    nw = nc * ns
    nblocks = VP // _LB
    per_w = nblocks // nw
    steps = per_w // kk
    groups = steps // 2
    bw = _LB * D
    ch = kk * bw
    mesh = plsc.VectorSubcoreMesh(core_axis_name="c", subcore_axis_name="s")

    @functools.partial(
        pl.kernel,
        out_type=jax.ShapeDtypeStruct((VP * D,), jnp.float32),
        mesh=mesh,
        scratch_types=(
            [pltpu.VMEM((kk, _LB * D), jnp.float32) for _ in range(2)]
            + [pltpu.VMEM((ch,), jnp.float32) for _ in range(2)]
            + [pltpu.SemaphoreType.DMA for _ in range(4)]
        ),
        compiler_params=pltpu.CompilerParams(
            use_tc_tiling_on_sc=False, needs_layout_passes=False
        ),
    )
    def format_kernel(t3_hbm, out_hbm, *scratch):
        in_v = scratch[:2]
        out_v = scratch[2:4]
        sem_i = scratch[4:6]
        sem_o = scratch[6:]
        wid = lax.axis_index("s") * nc + lax.axis_index("c")
        wbase = wid * per_w
        lane = lax.iota(jnp.int32, 16)
        lane8 = lane * D

        def body(g, _):
            k0s = [wbase + (g * 2 + b) * kk for b in range(2)]
            for b in range(2):
                @pl.when(g > 0)
                def _drain(b=b):
                    pltpu.make_async_copy(
                        out_v[b], out_hbm.at[pl.ds(0, ch)], sem_o[b]
                    ).wait()
                pltpu.async_copy(
                    t3_hbm.at[pl.ds(k0s[b], kk), :], in_v[b], sem_i[b]
                )
            for b in range(2):
                pltpu.make_async_copy(
                    t3_hbm.at[pl.ds(k0s[b], kk), :], in_v[b], sem_i[b]
                ).wait()

                def tbody(blk, _, b=b):
                    for m0 in range(0, bw, 16):
                        v = in_v[b][blk, pl.ds(m0, 16)]
                        d, c0 = m0 // _LB, m0 % _LB
                        idx = lane8 + (blk * bw + c0 * D + d)
                        plsc.store_scatter(out_v[b], [idx], v)
                    return 0

                lax.fori_loop(0, kk, tbody, 0, unroll=False)
                pltpu.async_copy(
                    out_v[b], out_hbm.at[pl.ds(k0s[b] * bw, ch)], sem_o[b]
                )
            return 0

        lax.fori_loop(0, groups, body, 0, unroll=False)
        for b in range(2):
            pltpu.make_async_copy(
                out_v[b], out_hbm.at[pl.ds(0, ch)], sem_o[b]
            ).wait()

    return format_kernel


def _make_gather(VP, D, B, nc, ns, ck):
    nw = nc * ns
    nblocks = B // _LB
    per_w = nblocks // nw
    steps = per_w // ck
    groups = steps // _NBUF
    ch = ck * _LB
    ob = ck * _LB * D
    mesh = plsc.VectorSubcoreMesh(core_axis_name="c", subcore_axis_name="s")

    @functools.partial(
        pl.kernel,
        out_type=jax.ShapeDtypeStruct((B * D,), jnp.float32),
        mesh=mesh,
        scratch_types=(
            [pltpu.VMEM((ch,), jnp.int32) for _ in range(_NBUF)]
            + [pltpu.VMEM((ch, D), jnp.float32) for _ in range(_NBUF)]
            + [pltpu.VMEM((ob,), jnp.float32) for _ in range(_NBUF)]
            + [pltpu.SemaphoreType.DMA for _ in range(2 * _NBUF)]
        ),
        compiler_params=pltpu.CompilerParams(
            use_tc_tiling_on_sc=False, needs_layout_passes=False
        ),
    )
    def gather_kernel(table_hbm, idx_hbm, out_hbm, *scratch):
        idx_v = scratch[:_NBUF]
        rows_v = scratch[_NBUF:2 * _NBUF]
        out_v = scratch[2 * _NBUF:3 * _NBUF]
        sem_g = scratch[3 * _NBUF:4 * _NBUF]
        sem_o = scratch[4 * _NBUF:]
        wid = lax.axis_index("s") * nc + lax.axis_index("c")
        wbase = wid * per_w
        lane = lax.iota(jnp.int32, 16)
        cols = [jnp.full((16,), d, jnp.int32) for d in range(D)]

        def body(g, _):
            k0s = [wbase + (g * _NBUF + b) * ck for b in range(_NBUF)]
            for b in range(_NBUF):
                @pl.when(g > 0)
                def _drain(b=b):
                    pltpu.make_async_copy(
                        out_v[b], out_hbm.at[pl.ds(0, ob)], sem_o[b]
                    ).wait()
                pltpu.sync_copy(idx_hbm.at[pl.ds(k0s[b] * _LB, ch)], idx_v[b])
                pltpu.async_copy(table_hbm.at[idx_v[b]], rows_v[b], sem_g[b])
            for b in range(_NBUF):
                pltpu.make_async_copy(
                    table_hbm.at[idx_v[b]], rows_v[b], sem_g[b]
                ).wait()

                def tbody(blk, _, b=b):
                    for g2 in range(_LB // 16):
                        row = lane + (blk * _LB + g2 * 16)
                        for d in range(D):
                            v = plsc.load_gather(rows_v[b], [row, cols[d]])
                            out_v[b][
                                pl.ds(blk * _LB * D + d * _LB + g2 * 16, 16)
                            ] = v
                    return 0

                lax.fori_loop(0, ck, tbody, 0, unroll=False)
                pltpu.async_copy(
                    out_v[b], out_hbm.at[pl.ds(k0s[b] * _LB * D, ob)], sem_o[b]
                )
            return 0

        lax.fori_loop(0, groups, body, 0, unroll=False)
        for b in range(_NBUF):
            pltpu.make_async_copy(
                out_v[b], out_hbm.at[pl.ds(0, ob)], sem_o[b]
            ).wait()

    return gather_kernel


def kernel(indices, table):
    B0, S = indices.shape
    V, D = table.shape
    B = B0 * S
    nb = B0 // _LB
    VP = 1 << 20
    info = plsc.get_sparse_core_info()
    nc, ns = info.num_cores, info.num_subcores

    tablep = jnp.pad(table, ((0, VP - V), (0, 0)))
    t3 = (
        tablep.T.reshape(D, VP // _LB, _LB)
        .transpose(1, 0, 2)
        .reshape(VP // _LB, _LB * D)
    )
    fmt = _make_format(D, VP, nc, ns, kk=16)
    t_rm = fmt(t3).reshape(VP, D)

    tidx = indices.T.reshape(B).astype(jnp.int32)
    gather = _make_gather(VP, D, B, nc, ns, ck=10)
    o = gather(t_rm, tidx)
    return (
        o.reshape(S, nb, D, _LB)
        .transpose(1, 3, 0, 2)
        .reshape(B0, S, D)
    )


# speedup vs baseline: 208.9373x; 1.7090x over previous
import functools

import jax
import jax.numpy as jnp
from jax import lax
from jax.experimental import pallas as pl
from jax.experimental.pallas import tpu as pltpu
from jax.experimental.pallas import tpu_sc as plsc

_NBUF = 4
_LB = 128


def _make_format(D, VP, nc, ns, kk):
    nw = nc * ns
    nblocks = VP // _LB
    per_w = nblocks // nw
    steps = per_w // kk
    groups = steps // 2
    bw = _LB * D
    ch = kk * bw
    mesh = plsc.VectorSubcoreMesh(core_axis_name="c", subcore_axis_name="s")

    @functools.partial(
        pl.kernel,
        out_type=jax.ShapeDtypeStruct((VP * D,), jnp.float32),
        mesh=mesh,
        scratch_types=(
            [pltpu.VMEM((kk, _LB * D), jnp.float32) for _ in range(2)]
            + [pltpu.VMEM((ch,), jnp.float32) for _ in range(2)]
            + [pltpu.SemaphoreType.DMA for _ in range(4)]
        ),
        compiler_params=pltpu.CompilerParams(
            use_tc_tiling_on_sc=False, needs_layout_passes=False
        ),
    )
    def format_kernel(t3_hbm, out_hbm, *scratch):
        in_v = scratch[:2]
        out_v = scratch[2:4]
        sem_i = scratch[4:6]
        sem_o = scratch[6:]
        wid = lax.axis_index("s") * nc + lax.axis_index("c")
        wbase = wid * per_w
        lane = lax.iota(jnp.int32, 16)
        lane8 = lane * D

        def body(g, _):
            k0s = [wbase + (g * 2 + b) * kk for b in range(2)]
            for b in range(2):
                @pl.when(g > 0)
                def _drain(b=b):
                    pltpu.make_async_copy(
                        out_v[b], out_hbm.at[pl.ds(0, ch)], sem_o[b]
                    ).wait()
                pltpu.async_copy(
                    t3_hbm.at[pl.ds(k0s[b], kk), :], in_v[b], sem_i[b]
                )
            for b in range(2):
                pltpu.make_async_copy(
                    t3_hbm.at[pl.ds(k0s[b], kk), :], in_v[b], sem_i[b]
                ).wait()

                def tbody(blk, _, b=b):
                    for half in range(0, bw, _LB):
                        vs = [
                            in_v[b][blk, pl.ds(half + t * 16, 16)]
                            for t in range(D)
                        ]
                        for t in range(D):
                            m0 = half + t * 16
                            d, c0 = m0 // _LB, m0 % _LB
                            idx = lane8 + (blk * bw + c0 * D + d)
                            plsc.store_scatter(out_v[b], [idx], vs[t])
                    return 0

                lax.fori_loop(0, kk, tbody, 0, unroll=False)
                pltpu.async_copy(
                    out_v[b], out_hbm.at[pl.ds(k0s[b] * bw, ch)], sem_o[b]
                )
            return 0

        lax.fori_loop(0, groups, body, 0, unroll=False)
        for b in range(2):
            pltpu.make_async_copy(
                out_v[b], out_hbm.at[pl.ds(0, ch)], sem_o[b]
            ).wait()

    return format_kernel


def _make_gather(VP, D, B, nc, ns, ck):
    nw = nc * ns
    nblocks = B // _LB
    per_w = nblocks // nw
    steps = per_w // ck
    groups = steps // _NBUF
    ch = ck * _LB
    ob = ck * _LB * D
    mesh = plsc.VectorSubcoreMesh(core_axis_name="c", subcore_axis_name="s")

    @functools.partial(
        pl.kernel,
        out_type=jax.ShapeDtypeStruct((B * D,), jnp.float32),
        mesh=mesh,
        scratch_types=(
            [pltpu.VMEM((ch,), jnp.int32) for _ in range(_NBUF)]
            + [pltpu.VMEM((ch, D), jnp.float32) for _ in range(_NBUF)]
            + [pltpu.VMEM((ob,), jnp.float32) for _ in range(_NBUF)]
            + [pltpu.SemaphoreType.DMA for _ in range(2 * _NBUF)]
        ),
        compiler_params=pltpu.CompilerParams(
            use_tc_tiling_on_sc=False, needs_layout_passes=False
        ),
    )
    def gather_kernel(table_hbm, idx_hbm, out_hbm, *scratch):
        idx_v = scratch[:_NBUF]
        rows_v = scratch[_NBUF:2 * _NBUF]
        out_v = scratch[2 * _NBUF:3 * _NBUF]
        sem_g = scratch[3 * _NBUF:4 * _NBUF]
        sem_o = scratch[4 * _NBUF:]
        wid = lax.axis_index("s") * nc + lax.axis_index("c")
        wbase = wid * per_w
        lane = lax.iota(jnp.int32, 16)
        cols = [jnp.full((16,), d, jnp.int32) for d in range(D)]

        def body(g, _):
            k0s = [wbase + (g * _NBUF + b) * ck for b in range(_NBUF)]
            for b in range(_NBUF):
                @pl.when(g > 0)
                def _drain(b=b):
                    pltpu.make_async_copy(
                        out_v[b], out_hbm.at[pl.ds(0, ob)], sem_o[b]
                    ).wait()
                pltpu.sync_copy(idx_hbm.at[pl.ds(k0s[b] * _LB, ch)], idx_v[b])
                pltpu.async_copy(table_hbm.at[idx_v[b]], rows_v[b], sem_g[b])
            for b in range(_NBUF):
                pltpu.make_async_copy(
                    table_hbm.at[idx_v[b]], rows_v[b], sem_g[b]
                ).wait()

                def tbody(blk, _, b=b):
                    for g2 in range(_LB // 16):
                        row = lane + (blk * _LB + g2 * 16)
                        vs = [
                            plsc.load_gather(rows_v[b], [row, cols[d]])
                            for d in range(D)
                        ]
                        for d in range(D):
                            out_v[b][
                                pl.ds(blk * _LB * D + d * _LB + g2 * 16, 16)
                            ] = vs[d]
                    return 0

                lax.fori_loop(0, ck, tbody, 0, unroll=False)
                pltpu.async_copy(
                    out_v[b], out_hbm.at[pl.ds(k0s[b] * _LB * D, ob)], sem_o[b]
                )
            return 0

        lax.fori_loop(0, groups, body, 0, unroll=False)
        for b in range(_NBUF):
            pltpu.make_async_copy(
                out_v[b], out_hbm.at[pl.ds(0, ob)], sem_o[b]
            ).wait()

    return gather_kernel


def kernel(indices, table):
    B0, S = indices.shape
    V, D = table.shape
    B = B0 * S
    nb = B0 // _LB
    VP = 1 << 20
    info = plsc.get_sparse_core_info()
    nc, ns = info.num_cores, info.num_subcores

    tablep = jnp.pad(table, ((0, VP - V), (0, 0)))
    t3 = (
        tablep.T.reshape(D, VP // _LB, _LB)
        .transpose(1, 0, 2)
        .reshape(VP // _LB, _LB * D)
    )
    fmt = _make_format(D, VP, nc, ns, kk=16)
    t_rm = fmt(t3).reshape(VP, D)

    tidx = indices.T.reshape(B).astype(jnp.int32)
    gather = _make_gather(VP, D, B, nc, ns, ck=10)
    o = gather(t_rm, tidx)
    return (
        o.reshape(S, nb, D, _LB)
        .transpose(1, 3, 0, 2)
        .reshape(B0, S, D)
    )
